# baseline (speedup 1.0000x reference)
import functools

import jax
import jax.numpy as jnp
from jax import lax
from jax.experimental import pallas as pl
from jax.experimental.pallas import tpu as pltpu
from jax.experimental.pallas import tpu_sc as plsc

N = 10000
E = 160000
D = 128
H = 64
NO = 4
RD = 8
AD = 9

NC = 2
NS = 16
NW = NC * NS

EPAD = 163840
NSPLIT = 2
EH = EPAD // NSPLIT
RPT = EH // NW

SUB = 128
SSUP = 640
GSUP = 512
GSUB = GSUP // SUB
NP = 10240
EC = 32
RPC = NP // NS
BE = 2048

_f32 = jnp.float32
_bf16 = jnp.bfloat16


def _silu(x):
    return x * jax.nn.sigmoid(x)


def _lrelu(x):
    return jnp.where(x >= 0, x, 0.01 * x)


def _bdot(x, w):
    return jnp.dot(x.astype(_bf16), w.astype(_bf16),
                   preferred_element_type=_f32)


def _node_proj_body(ne, ws, wd, o):
    x = ne[...]
    a = jnp.dot(x, ws[...], preferred_element_type=_f32)
    b = jnp.dot(x, wd[...], preferred_element_type=_f32)
    o[...] = jnp.concatenate([a, b], axis=1).astype(_bf16)


def _node_proj(ne, ws, wd):
    bn = 1000
    return pl.pallas_call(
        _node_proj_body,
        grid=(N // bn,),
        in_specs=[
            pl.BlockSpec((bn, D), lambda i: (i, 0)),
            pl.BlockSpec((D, H), lambda i: (0, 0)),
            pl.BlockSpec((D, H), lambda i: (0, 0)),
        ],
        out_specs=pl.BlockSpec((bn, D), lambda i: (i, 0)),
        out_shape=jax.ShapeDtypeStruct((N, D), _bf16),
    )(ne, ws, wd)


def _edge_mlp_body(g1, g2, ra, aa, w1r, w1a, b1, w2, b2, w3, b3, w4, b4,
                   w5, b5, w6, b6, out):
    h = _silu(g1[:, :H].astype(_f32) + g2[:, H:].astype(_f32)
              + _bdot(ra[...], w1r[...]) + _bdot(aa[...], w1a[...])
              + b1[...])
    h = _silu(_bdot(h, w2[...]) + b2[...])
    h = _silu(_bdot(h, w3[...]) + b3[...])
    h = _silu(_bdot(h, w4[...]) + b4[...])
    h = _lrelu(_bdot(h, w5[...]) + b5[...])
    out[...] = _bdot(h, w6[...]) + b6[...]


def _edge_mlp(g1, g2, ra, aa, hb, w1r, w1a, b1, w2, b2, w3, b3, w4, b4,
              w5, b5, w6, b6):
    full = lambda a, b: pl.BlockSpec((a, b), lambda i: (0, 0))
    return pl.pallas_call(
        _edge_mlp_body,
        grid=(EH // BE,),
        in_specs=[
            pl.BlockSpec((BE, D), lambda i: (i, 0)),
            pl.BlockSpec((BE, D), lambda i: (i, 0)),
            pl.BlockSpec((BE, RD), lambda i, hb=hb: (i + hb, 0)),
            pl.BlockSpec((BE, AD), lambda i, hb=hb: (i + hb, 0)),
            full(RD, H), full(AD, H), full(1, H),
            full(H, H), full(1, H),
            full(H, H // 2), full(1, H // 2),
            full(H // 2, H), full(1, H),
            full(H, H), full(1, H),
            full(H, EC), full(1, EC),
        ],
        out_specs=pl.BlockSpec((BE, EC), lambda i: (i, 0)),
        out_shape=jax.ShapeDtypeStruct((EH, EC), _f32),
    )(g1, g2, ra, aa, w1r, w1a, b1, w2, b2, w3, b3, w4, b4, w5, b5, w6, b6)


def _node_update_body(ne, agg_refs, wn, wa, b1, w2, b2, hs, hd, out):
    a = agg_refs[0][0] + agg_refs[0][1]
    for r in agg_refs[1:]:
        a = a + r[0] + r[1]
    h = _silu(_bdot(ne[...], wn[...]) + _bdot(a, wa[...]) + b1[...])
    nu = _bdot(h, w2[...]) + b2[...]
    p1 = _bdot(nu, hs[...])
    p2 = _bdot(nu, hd[...])
    out[...] = jnp.concatenate([p1, p2], axis=1).astype(_bf16)


def _node_update(ne, aggs, wn, wa, b1, w2, b2, hs, hd):
    bn = 1000
    full = lambda a, b: pl.BlockSpec((a, b), lambda i: (0, 0))

    def body(ne_, *rest):
        agg_refs = rest[:NSPLIT]
        (wn_, wa_, b1_, w2_, b2_, hs_, hd_, out_) = rest[NSPLIT:]
        _node_update_body(ne_, agg_refs, wn_, wa_, b1_, w2_, b2_, hs_, hd_,
                          out_)

    return pl.pallas_call(
        body,
        grid=(N // bn,),
        in_specs=[pl.BlockSpec((bn, D), lambda i: (i, 0))]
        + [pl.BlockSpec((NC, bn, EC), lambda i: (0, i, 0))] * NSPLIT
        + [full(D, H), full(EC, H), full(1, H),
           full(H, D), full(1, D),
           full(D, H), full(D, H)],
        out_specs=pl.BlockSpec((bn, D), lambda i: (i, 0)),
        out_shape=jax.ShapeDtypeStruct((N, D), _bf16),
    )(ne, *aggs, wn, wa, b1, w2, b2, hs, hd)


def _head_body(g1, g2, ef, w1, b1, w2, b2, w3, b3, w4, b4, w5, b5, out):
    h = _silu(g1[:, :H].astype(_f32) + g2[:, H:].astype(_f32)
              + _bdot(ef[...], w1[...]) + b1[...])
    h = _silu(_bdot(h, w2[...]) + b2[...])
    h = _silu(_bdot(h, w3[...]) + b3[...])
    h = _lrelu(_bdot(h, w4[...]) + b4[...])
    out[...] = _bdot(h, w5[...]) + b5[...]


def _head_mlp(g1, g2, ef, w1, b1, w2, b2, w3, b3, w4, b4, w5, b5):
    no2 = NO * NO
    full = lambda a, b: pl.BlockSpec((a, b), lambda i: (0, 0))
    return pl.pallas_call(
        _head_body,
        grid=(EH // BE,),
        in_specs=[
            pl.BlockSpec((BE, D), lambda i: (i, 0)),
            pl.BlockSpec((BE, D), lambda i: (i, 0)),
            pl.BlockSpec((BE, EC), lambda i: (i, 0)),
            full(EC, H), full(1, H),
            full(H, H), full(1, H),
            full(H, H), full(1, H),
            full(H, H), full(1, H),
            full(H, no2), full(1, no2),
        ],
        out_specs=pl.BlockSpec((BE, no2), lambda i: (i, 0)),
        out_shape=jax.ShapeDtypeStruct((EH, no2), _f32),
    )(g1, g2, ef, w1, b1, w2, b2, w3, b3, w4, b4, w5, b5)


def _make_gather2_body(ho):
    def body(tab, ia, ib, oa, ob, iva, ivb, buf0, buf1,
             gsem0, gsem1, osem0, osem1):
        c = lax.axis_index("c")
        s = lax.axis_index("s")
        base = (c * NS + s) * RPT
        pltpu.sync_copy(ia.at[pl.ds(ho + base, RPT)], iva)
        pltpu.sync_copy(ib.at[pl.ds(ho + base, RPT)], ivb)

        bufs = (buf0, buf1)
        gsems = (gsem0, gsem1)
        osems = (osem0, osem1)
        ngsup = RPT // GSUP
        chunks = [(iva, oa, u) for u in range(ngsup)] + \
                 [(ivb, ob, u) for u in range(ngsup)]
        nk = len(chunks)
        ghandles = {}
        ohandles = {}

        def flush(k):
            pb = k & 1
            for h in ghandles[k]:
                h.wait()
            _, pout, pu = chunks[k]
            ohandles[k] = pltpu.async_copy(
                bufs[pb], pout.at[pl.ds(base + pu * GSUP, GSUP)], osems[pb])

        for k, (iv, out, u) in enumerate(chunks):
            b = k & 1
            if k >= 2:
                ohandles[k - 2].wait()
            offs = u * GSUP
            ghandles[k] = [
                pltpu.async_copy(
                    tab.at[iv.at[pl.ds(offs + j * SUB, SUB)]],
                    bufs[b].at[pl.ds(j * SUB, SUB)], gsems[b])
                for j in range(GSUB)
            ]
            if k >= 1:
                flush(k - 1)
        flush(nk - 1)
        ohandles[nk - 2].wait()
        ohandles[nk - 1].wait()

    return body


def _gather2(tab, ia, ib, ho):
    mesh = plsc.VectorSubcoreMesh(core_axis_name="c", subcore_axis_name="s")
    k = pl.kernel(
        _make_gather2_body(ho),
        out_type=(jax.ShapeDtypeStruct((EH, D), _bf16),
                  jax.ShapeDtypeStruct((EH, D), _bf16)),
        mesh=mesh,
        compiler_params=pltpu.CompilerParams(use_tc_tiling_on_sc=False),
        scratch_types=[
            pltpu.VMEM((RPT,), jnp.int32),
            pltpu.VMEM((RPT,), jnp.int32),
            pltpu.VMEM((GSUP, D), _bf16),
            pltpu.VMEM((GSUP, D), _bf16),
            pltpu.SemaphoreType.DMA,
            pltpu.SemaphoreType.DMA,
            pltpu.SemaphoreType.DMA,
            pltpu.SemaphoreType.DMA,
        ],
    )
    return k(tab, ia, ib)


def _make_scatter_body(ho):
    nsub = SSUP // SUB
    nsup = RPT // SSUP
    nchk = RPT // SUB

    def body(eu, idx2, zero, out, idxv, rows, shared):
        c = lax.axis_index("c")
        s = lax.axis_index("s")
        wid = c * NS + s
        pltpu.sync_copy(zero.at[pl.ds(s * RPC, RPC)],
                        shared.at[pl.ds(s * RPC, RPC)])
        pltpu.sync_copy(idx2.at[pl.ds(ho // SUB + wid * nchk, nchk)], idxv)
        plsc.subcore_barrier()

        def step(u, carry):
            pltpu.sync_copy(eu.at[pl.ds(wid * RPT + u * SSUP, SSUP)], rows)
            for j in range(nsub):
                pltpu.sync_copy(rows.at[pl.ds(j * SUB, SUB)],
                                shared.at[idxv.at[u * nsub + j]], add=True)
            return carry

        lax.fori_loop(0, nsup, step, 0)
        plsc.subcore_barrier()
        pltpu.sync_copy(shared.at[pl.ds(s * RPC, RPC)],
                        out.at[pl.ds(c * NP + s * RPC, RPC)])

    return body


def _scatter_add(eu, idx2, zero, ho):
    mesh = plsc.VectorSubcoreMesh(core_axis_name="c", subcore_axis_name="s")
    k = pl.kernel(
        _make_scatter_body(ho),
        out_type=jax.ShapeDtypeStruct((NC * NP, EC), _f32),
        mesh=mesh,
        compiler_params=pltpu.CompilerParams(use_tc_tiling_on_sc=False),
        scratch_types=[
            pltpu.VMEM((RPT // SUB, SUB), jnp.int32),
            pltpu.VMEM((SSUP, EC), _f32),
            pltpu.VMEM_SHARED((NP, EC), _f32),
        ],
    )
    return k(eu, idx2, zero)


def kernel(node_env, edge_radial, edge_angular, edge_index,
           nu_w1, nu_b1, nu_w2, nu_b2,
           eu_w1, eu_b1, eu_w2, eu_b2, eu_w3, eu_b3, eu_w4, eu_b4,
           eu_w5, eu_b5, eu_w6, eu_b6,
           h_w1, h_b1, h_w2, h_b2, h_w3, h_b3, h_w4, h_b4, h_w5, h_b5):
    pad_e = EPAD - E

    src = jnp.concatenate([edge_index[0],
                           jnp.zeros((pad_e,), jnp.int32)])
    dst = jnp.concatenate([edge_index[1],
                           jnp.full((pad_e,), N, jnp.int32)])
    dst2 = dst.reshape(EPAD // SUB, SUB)

    ra = jnp.concatenate([edge_radial, jnp.zeros((pad_e, RD), _f32)], axis=0)
    aa = jnp.concatenate([edge_angular, jnp.zeros((pad_e, AD), _f32)], axis=0)

    def pad_rows(w):
        return jnp.concatenate(
            [w, jnp.zeros((EC - w.shape[0], w.shape[1]), _f32)], axis=0)

    eu1s, eu1d = eu_w1[:D], eu_w1[D:2 * D]
    eu1r = eu_w1[2 * D:2 * D + RD]
    eu1a = eu_w1[2 * D + RD:]
    h1s, h1d = h_w1[:D], h_w1[D:2 * D]
    h1e = pad_rows(h_w1[2 * D:])
    nu1n = nu_w1[:D]
    nu1a = pad_rows(nu_w1[D:])
    eu_w6p = jnp.concatenate(
        [eu_w6, jnp.zeros((H, EC - eu_w6.shape[1]), _f32)], axis=1)
    eu_b6p = jnp.concatenate([eu_b6, jnp.zeros((EC - eu_b6.shape[0],), _f32)])

    r1 = lambda b: b.reshape(1, -1)
    zero_np = jnp.zeros((NP, EC), _f32)
    hos = [i * EH for i in range(NSPLIT)]
    hbs = [ho // BE for ho in hos]

    t1 = _node_proj(node_env, eu1s, eu1d)
    gs = [_gather2(t1, src, dst, ho) for ho in hos]
    e_upds = [_edge_mlp(gs[i][0], gs[i][1], ra, aa, hbs[i],
                        eu1r, eu1a, r1(eu_b1), eu_w2, r1(eu_b2),
                        eu_w3, r1(eu_b3), eu_w4, r1(eu_b4), eu_w5, r1(eu_b5),
                        eu_w6p, r1(eu_b6p))
              for i in range(NSPLIT)]
    aggs = [_scatter_add(e_upds[i], dst2, zero_np, hos[i]).reshape(NC, NP, EC)
            for i in range(NSPLIT)]
    t2 = _node_update(node_env, aggs,
                      nu1n, nu1a, r1(nu_b1), nu_w2, r1(nu_b2), h1s, h1d)
    g2s = [_gather2(t2, src, dst, ho) for ho in hos]
    outs = [_head_mlp(g2s[i][0], g2s[i][1], e_upds[i],
                      h1e, r1(h_b1), h_w2, r1(h_b2), h_w3, r1(h_b3),
                      h_w4, r1(h_b4), h_w5, r1(h_b5))
            for i in range(NSPLIT)]
    out = jnp.concatenate(outs, axis=0)
    return out[:E].reshape(E, NO, NO)


# speedup vs baseline: 1.3768x; 1.3768x over previous
import functools

import jax
import jax.numpy as jnp
from jax import lax
from jax.experimental import pallas as pl
from jax.experimental.pallas import tpu as pltpu
from jax.experimental.pallas import tpu_sc as plsc

N = 10000
E = 160000
D = 128
H = 64
NO = 4

NC = 2
NS = 16
NW = NC * NS

EPAD = 163840
NSPLIT = 2
EH = EPAD // NSPLIT

SUB = 128
SSUP = 640
GSUP = 640
GSUB = GSUP // SUB
NP = 10240
EC = 32
RPC = NP // NS

_f32 = jnp.float32
_bf16 = jnp.bfloat16


def _silu(x):
    return x * jax.nn.sigmoid(x)


def _lrelu(x):
    return jnp.where(x >= 0, x, 0.01 * x)


def _bdot(x, w):
    return jnp.dot(x.astype(_bf16), w.astype(_bf16),
                   preferred_element_type=_f32)


def _node_proj_body(ne, ws, wd, o1, o2):
    x = ne[...]
    o1[...] = jnp.dot(x, ws[...], preferred_element_type=_f32).astype(_bf16)
    o2[...] = jnp.dot(x, wd[...], preferred_element_type=_f32).astype(_bf16)


def _node_proj(ne, ws, wd):
    bn = 1000
    return pl.pallas_call(
        _node_proj_body,
        grid=(N // bn,),
        in_specs=[
            pl.BlockSpec((bn, D), lambda i: (i, 0)),
            pl.BlockSpec((D, H), lambda i: (0, 0)),
            pl.BlockSpec((D, H), lambda i: (0, 0)),
        ],
        out_specs=[pl.BlockSpec((bn, H), lambda i: (i, 0))] * 2,
        out_shape=[jax.ShapeDtypeStruct((N, H), _bf16)] * 2,
    )(ne, ws, wd)


def _edge_mlp_body(g1, g2, ef, w1, b1, w2, b2, w3, b3, w4, b4, w5, b5,
                   w6, b6, out):
    h = _silu(g1[...].astype(_f32) + g2[...].astype(_f32)
              + _bdot(ef[...], w1[...]) + b1[...])
    h = _silu(_bdot(h, w2[...]) + b2[...])
    h = _silu(_bdot(h, w3[...]) + b3[...])
    h = _silu(_bdot(h, w4[...]) + b4[...])
    h = _lrelu(_bdot(h, w5[...]) + b5[...])
    out[...] = _bdot(h, w6[...]) + b6[...]


def _edge_mlp(g1, g2, ef, w1, b1, w2, b2, w3, b3, w4, b4, w5, b5, w6, b6):
    nrows = g1.shape[0]
    be = 2048
    full = lambda a, b: pl.BlockSpec((a, b), lambda i: (0, 0))
    return pl.pallas_call(
        _edge_mlp_body,
        grid=(nrows // be,),
        in_specs=[
            pl.BlockSpec((be, H), lambda i: (i, 0)),
            pl.BlockSpec((be, H), lambda i: (i, 0)),
            pl.BlockSpec((be, EC), lambda i: (i, 0)),
            full(EC, H), full(1, H),
            full(H, H), full(1, H),
            full(H, H // 2), full(1, H // 2),
            full(H // 2, H), full(1, H),
            full(H, H), full(1, H),
            full(H, EC), full(1, EC),
        ],
        out_specs=pl.BlockSpec((be, EC), lambda i: (i, 0)),
        out_shape=jax.ShapeDtypeStruct((nrows, EC), _f32),
    )(g1, g2, ef, w1, b1, w2, b2, w3, b3, w4, b4, w5, b5, w6, b6)


def _node_update_body(ne, agg_refs, wn, wa, b1, w2, b2, hs, hd, o1, o2):
    a = agg_refs[0][0] + agg_refs[0][1]
    for r in agg_refs[1:]:
        a = a + r[0] + r[1]
    h = _silu(jnp.dot(ne[...], wn[...], preferred_element_type=_f32)
              + jnp.dot(a, wa[...], preferred_element_type=_f32)
              + b1[...])
    nu = jnp.dot(h, w2[...], preferred_element_type=_f32) + b2[...]
    o1[...] = jnp.dot(nu, hs[...], preferred_element_type=_f32).astype(_bf16)
    o2[...] = jnp.dot(nu, hd[...], preferred_element_type=_f32).astype(_bf16)


def _node_update(ne, aggs, wn, wa, b1, w2, b2, hs, hd):
    bn = 1000
    full = lambda a, b: pl.BlockSpec((a, b), lambda i: (0, 0))

    def body(ne_, *rest):
        agg_refs = rest[:NSPLIT]
        (wn_, wa_, b1_, w2_, b2_, hs_, hd_, o1_, o2_) = rest[NSPLIT:]
        _node_update_body(ne_, agg_refs, wn_, wa_, b1_, w2_, b2_, hs_, hd_,
                          o1_, o2_)

    return pl.pallas_call(
        body,
        grid=(N // bn,),
        in_specs=[pl.BlockSpec((bn, D), lambda i: (i, 0))]
        + [pl.BlockSpec((NC, bn, EC), lambda i: (0, i, 0))] * NSPLIT
        + [full(D, H), full(EC, H), full(1, H),
           full(H, D), full(1, D),
           full(D, H), full(D, H)],
        out_specs=[pl.BlockSpec((bn, H), lambda i: (i, 0))] * 2,
        out_shape=[jax.ShapeDtypeStruct((N, H), _bf16)] * 2,
    )(ne, *aggs, wn, wa, b1, w2, b2, hs, hd)


def _head_body(g1, g2, ef, w1, b1, w2, b2, w3, b3, w4, b4, w5, b5, out):
    h = _silu(g1[...].astype(_f32) + g2[...].astype(_f32)
              + _bdot(ef[...], w1[...]) + b1[...])
    h = _silu(_bdot(h, w2[...]) + b2[...])
    h = _silu(_bdot(h, w3[...]) + b3[...])
    h = _lrelu(_bdot(h, w4[...]) + b4[...])
    out[...] = _bdot(h, w5[...]) + b5[...]


def _head_mlp(g1, g2, ef, w1, b1, w2, b2, w3, b3, w4, b4, w5, b5):
    nrows = g1.shape[0]
    be = 2048
    no2 = NO * NO
    full = lambda a, b: pl.BlockSpec((a, b), lambda i: (0, 0))
    return pl.pallas_call(
        _head_body,
        grid=(nrows // be,),
        in_specs=[
            pl.BlockSpec((be, H), lambda i: (i, 0)),
            pl.BlockSpec((be, H), lambda i: (i, 0)),
            pl.BlockSpec((be, EC), lambda i: (i, 0)),
            full(EC, H), full(1, H),
            full(H, H), full(1, H),
            full(H, H), full(1, H),
            full(H, H), full(1, H),
            full(H, no2), full(1, no2),
        ],
        out_specs=pl.BlockSpec((be, no2), lambda i: (i, 0)),
        out_shape=jax.ShapeDtypeStruct((nrows, no2), _f32),
    )(g1, g2, ef, w1, b1, w2, b2, w3, b3, w4, b4, w5, b5)


def _make_gather2_body(rpt):
    ngsup = rpt // GSUP

    def body(ta, tb, ia, ib, oa, ob, iva, ivb, buf0, buf1,
             gsem0, gsem1, osem0, osem1):
        c = lax.axis_index("c")
        s = lax.axis_index("s")
        base = (c * NS + s) * rpt
        pltpu.sync_copy(ia.at[pl.ds(base, rpt)], iva)
        pltpu.sync_copy(ib.at[pl.ds(base, rpt)], ivb)

        bufs = (buf0, buf1)
        gsems = (gsem0, gsem1)
        osems = (osem0, osem1)
        chunks = [(ta, iva, oa, u) for u in range(ngsup)] + \
                 [(tb, ivb, ob, u) for u in range(ngsup)]
        nk = len(chunks)
        ghandles = {}
        ohandles = {}

        def flush(k):
            pb = k & 1
            for h in ghandles[k]:
                h.wait()
            _, _, pout, pu = chunks[k]
            ohandles[k] = pltpu.async_copy(
                bufs[pb], pout.at[pl.ds(base + pu * GSUP, GSUP)], osems[pb])

        for k, (tab, iv, out, u) in enumerate(chunks):
            b = k & 1
            if k >= 2:
                ohandles[k - 2].wait()
            offs = u * GSUP
            ghandles[k] = [
                pltpu.async_copy(
                    tab.at[iv.at[pl.ds(offs + j * SUB, SUB)]],
                    bufs[b].at[pl.ds(j * SUB, SUB)], gsems[b])
                for j in range(GSUB)
            ]
            if k >= 1:
                flush(k - 1)
        flush(nk - 1)
        ohandles[nk - 2].wait()
        ohandles[nk - 1].wait()

    return body


def _gather2(ta, tb, ia, ib):
    nrows = ia.shape[0]
    rpt = nrows // NW
    mesh = plsc.VectorSubcoreMesh(core_axis_name="c", subcore_axis_name="s")
    k = pl.kernel(
        _make_gather2_body(rpt),
        out_type=(jax.ShapeDtypeStruct((nrows, H), _bf16),
                  jax.ShapeDtypeStruct((nrows, H), _bf16)),
        mesh=mesh,
        compiler_params=pltpu.CompilerParams(use_tc_tiling_on_sc=False),
        scratch_types=[
            pltpu.VMEM((rpt,), jnp.int32),
            pltpu.VMEM((rpt,), jnp.int32),
            pltpu.VMEM((GSUP, H), _bf16),
            pltpu.VMEM((GSUP, H), _bf16),
            pltpu.SemaphoreType.DMA,
            pltpu.SemaphoreType.DMA,
            pltpu.SemaphoreType.DMA,
            pltpu.SemaphoreType.DMA,
        ],
    )
    return k(ta, tb, ia, ib)


def _make_scatter_body(rpt):
    nsub = SSUP // SUB
    nsup = rpt // SSUP
    nchk = rpt // SUB

    def body(eu, idx2, zero, out, idxv, rows, shared):
        c = lax.axis_index("c")
        s = lax.axis_index("s")
        wid = c * NS + s
        pltpu.sync_copy(zero.at[pl.ds(s * RPC, RPC)],
                        shared.at[pl.ds(s * RPC, RPC)])
        pltpu.sync_copy(idx2.at[pl.ds(wid * nchk, nchk)], idxv)
        plsc.subcore_barrier()

        def step(u, carry):
            pltpu.sync_copy(eu.at[pl.ds(wid * rpt + u * SSUP, SSUP)], rows)
            for j in range(nsub):
                pltpu.sync_copy(rows.at[pl.ds(j * SUB, SUB)],
                                shared.at[idxv.at[u * nsub + j]], add=True)
            return carry

        lax.fori_loop(0, nsup, step, 0)
        plsc.subcore_barrier()
        pltpu.sync_copy(shared.at[pl.ds(s * RPC, RPC)],
                        out.at[pl.ds(c * NP + s * RPC, RPC)])

    return body


def _scatter_add(eu, idx2, zero):
    rpt = eu.shape[0] // NW
    mesh = plsc.VectorSubcoreMesh(core_axis_name="c", subcore_axis_name="s")
    k = pl.kernel(
        _make_scatter_body(rpt),
        out_type=jax.ShapeDtypeStruct((NC * NP, EC), _f32),
        mesh=mesh,
        compiler_params=pltpu.CompilerParams(use_tc_tiling_on_sc=False),
        scratch_types=[
            pltpu.VMEM((rpt // SUB, SUB), jnp.int32),
            pltpu.VMEM((SSUP, EC), _f32),
            pltpu.VMEM_SHARED((NP, EC), _f32),
        ],
    )
    return k(eu, idx2, zero)


def kernel(node_env, edge_radial, edge_angular, edge_index,
           nu_w1, nu_b1, nu_w2, nu_b2,
           eu_w1, eu_b1, eu_w2, eu_b2, eu_w3, eu_b3, eu_w4, eu_b4,
           eu_w5, eu_b5, eu_w6, eu_b6,
           h_w1, h_b1, h_w2, h_b2, h_w3, h_b3, h_w4, h_b4, h_w5, h_b5):
    rd = edge_radial.shape[1]
    ed = rd + edge_angular.shape[1]
    pad_e = EPAD - E

    src = jnp.concatenate([edge_index[0],
                           jnp.zeros((pad_e,), jnp.int32)])
    dst = jnp.concatenate([edge_index[1],
                           jnp.full((pad_e,), N, jnp.int32)])

    ef = jnp.concatenate(
        [edge_radial, edge_angular,
         jnp.zeros((E, EC - ed), _f32)], axis=1)
    ef = jnp.concatenate([ef, jnp.zeros((pad_e, EC), _f32)], axis=0)

    def pad_rows(w):
        return jnp.concatenate(
            [w, jnp.zeros((EC - w.shape[0], w.shape[1]), _f32)], axis=0)

    eu1s, eu1d = eu_w1[:D], eu_w1[D:2 * D]
    eu1e = pad_rows(eu_w1[2 * D:])
    h1s, h1d = h_w1[:D], h_w1[D:2 * D]
    h1e = pad_rows(h_w1[2 * D:])
    nu1n = nu_w1[:D]
    nu1a = pad_rows(nu_w1[D:])
    eu_w6p = jnp.concatenate(
        [eu_w6, jnp.zeros((H, EC - eu_w6.shape[1]), _f32)], axis=1)
    eu_b6p = jnp.concatenate([eu_b6, jnp.zeros((EC - eu_b6.shape[0],), _f32)])

    r1 = lambda b: b.reshape(1, -1)
    zero_np = jnp.zeros((NP, EC), _f32)

    srcs = [lax.slice(src, (i * EH,), ((i + 1) * EH,)) for i in range(NSPLIT)]
    dsts = [lax.slice(dst, (i * EH,), ((i + 1) * EH,)) for i in range(NSPLIT)]
    dst2s = [d.reshape(EH // SUB, SUB) for d in dsts]
    efs = [lax.slice(ef, (i * EH, 0), ((i + 1) * EH, EC))
           for i in range(NSPLIT)]

    p1s, p1d = _node_proj(node_env, eu1s, eu1d)
    gs = [_gather2(p1s, p1d, srcs[i], dsts[i]) for i in range(NSPLIT)]
    e_upds = [_edge_mlp(gs[i][0], gs[i][1], efs[i],
                        eu1e, r1(eu_b1), eu_w2, r1(eu_b2), eu_w3, r1(eu_b3),
                        eu_w4, r1(eu_b4), eu_w5, r1(eu_b5),
                        eu_w6p, r1(eu_b6p))
              for i in range(NSPLIT)]
    aggs = [_scatter_add(e_upds[i], dst2s[i], zero_np).reshape(NC, NP, EC)
            for i in range(NSPLIT)]
    p2s, p2d = _node_update(node_env, aggs,
                            nu1n, nu1a, r1(nu_b1), nu_w2, r1(nu_b2),
                            h1s, h1d)
    g2s = [_gather2(p2s, p2d, srcs[i], dsts[i]) for i in range(NSPLIT)]
    outs = [_head_mlp(g2s[i][0], g2s[i][1], e_upds[i],
                      h1e, r1(h_b1), h_w2, r1(h_b2), h_w3, r1(h_b3),
                      h_w4, r1(h_b4), h_w5, r1(h_b5))
            for i in range(NSPLIT)]
    out = jnp.concatenate(outs, axis=0)
    return out[:E].reshape(E, NO, NO)


# speedup vs baseline: 1.3996x; 1.0166x over previous
import functools

import jax
import jax.numpy as jnp
from jax import lax
from jax.experimental import pallas as pl
from jax.experimental.pallas import tpu as pltpu
from jax.experimental.pallas import tpu_sc as plsc

N = 10000
E = 160000
D = 128
H = 64
NO = 4

NC = 2
NS = 16
NW = NC * NS

EPAD = 163840
NSPLIT = 2
EH = EPAD // NSPLIT

SUB = 128
SSUP = 640
GSUP = 640
GSUB = GSUP // SUB
NP = 10240
EC = 32
RPC = NP // NS

_f32 = jnp.float32
_bf16 = jnp.bfloat16


def _silu(x):
    return x * jax.nn.sigmoid(x)


def _lrelu(x):
    return jnp.where(x >= 0, x, 0.01 * x)


def _bdot(x, w):
    return jnp.dot(x.astype(_bf16), w.astype(_bf16),
                   preferred_element_type=_f32)


def _node_proj_body(ne, ws, wd, o1, o2):
    x = ne[...]
    o1[...] = jnp.dot(x, ws[...], preferred_element_type=_f32).astype(_bf16)
    o2[...] = jnp.dot(x, wd[...], preferred_element_type=_f32).astype(_bf16)


def _node_proj(ne, ws, wd):
    bn = 1000
    return pl.pallas_call(
        _node_proj_body,
        grid=(N // bn,),
        in_specs=[
            pl.BlockSpec((bn, D), lambda i: (i, 0)),
            pl.BlockSpec((D, H), lambda i: (0, 0)),
            pl.BlockSpec((D, H), lambda i: (0, 0)),
        ],
        out_specs=[pl.BlockSpec((bn, H), lambda i: (i, 0))] * 2,
        out_shape=[jax.ShapeDtypeStruct((N, H), _bf16)] * 2,
    )(ne, ws, wd)


def _edge_mlp_body(g1, g2, ef, w1, b1, w2, b2, w3, b3, w4, b4, w5, b5,
                   w6, b6, out):
    h = _silu(g1[...].astype(_f32) + g2[...].astype(_f32)
              + _bdot(ef[...], w1[...]) + b1[...])
    h = _silu(_bdot(h, w2[...]) + b2[...])
    h = _silu(_bdot(h, w3[...]) + b3[...])
    h = _silu(_bdot(h, w4[...]) + b4[...])
    h = _lrelu(_bdot(h, w5[...]) + b5[...])
    out[...] = _bdot(h, w6[...]) + b6[...]


def _edge_mlp(g1, g2, ef, hb, w1, b1, w2, b2, w3, b3, w4, b4, w5, b5,
              w6, b6):
    nrows = g1.shape[0]
    be = 2048
    full = lambda a, b: pl.BlockSpec((a, b), lambda i: (0, 0))
    return pl.pallas_call(
        _edge_mlp_body,
        grid=(nrows // be,),
        in_specs=[
            pl.BlockSpec((be, H), lambda i: (i, 0)),
            pl.BlockSpec((be, H), lambda i: (i, 0)),
            pl.BlockSpec((be, EC), lambda i, hb=hb: (i + hb, 0)),
            full(EC, H), full(1, H),
            full(H, H), full(1, H),
            full(H, H // 2), full(1, H // 2),
            full(H // 2, H), full(1, H),
            full(H, H), full(1, H),
            full(H, EC), full(1, EC),
        ],
        out_specs=pl.BlockSpec((be, EC), lambda i: (i, 0)),
        out_shape=jax.ShapeDtypeStruct((nrows, EC), _f32),
    )(g1, g2, ef, w1, b1, w2, b2, w3, b3, w4, b4, w5, b5, w6, b6)


def _node_update_body(ne, agg_refs, wn, wa, b1, w2, b2, hs, hd, o1, o2):
    a = agg_refs[0][0] + agg_refs[0][1]
    for r in agg_refs[1:]:
        a = a + r[0] + r[1]
    h = _silu(jnp.dot(ne[...], wn[...], preferred_element_type=_f32)
              + jnp.dot(a, wa[...], preferred_element_type=_f32)
              + b1[...])
    nu = jnp.dot(h, w2[...], preferred_element_type=_f32) + b2[...]
    o1[...] = jnp.dot(nu, hs[...], preferred_element_type=_f32).astype(_bf16)
    o2[...] = jnp.dot(nu, hd[...], preferred_element_type=_f32).astype(_bf16)


def _node_update(ne, aggs, wn, wa, b1, w2, b2, hs, hd):
    bn = 1000
    full = lambda a, b: pl.BlockSpec((a, b), lambda i: (0, 0))

    def body(ne_, *rest):
        agg_refs = rest[:NSPLIT]
        (wn_, wa_, b1_, w2_, b2_, hs_, hd_, o1_, o2_) = rest[NSPLIT:]
        _node_update_body(ne_, agg_refs, wn_, wa_, b1_, w2_, b2_, hs_, hd_,
                          o1_, o2_)

    return pl.pallas_call(
        body,
        grid=(N // bn,),
        in_specs=[pl.BlockSpec((bn, D), lambda i: (i, 0))]
        + [pl.BlockSpec((NC, bn, EC), lambda i: (0, i, 0))] * NSPLIT
        + [full(D, H), full(EC, H), full(1, H),
           full(H, D), full(1, D),
           full(D, H), full(D, H)],
        out_specs=[pl.BlockSpec((bn, H), lambda i: (i, 0))] * 2,
        out_shape=[jax.ShapeDtypeStruct((N, H), _bf16)] * 2,
    )(ne, *aggs, wn, wa, b1, w2, b2, hs, hd)


def _head_body(g1, g2, ef, w1, b1, w2, b2, w3, b3, w4, b4, w5, b5, out):
    h = _silu(g1[...].astype(_f32) + g2[...].astype(_f32)
              + _bdot(ef[...], w1[...]) + b1[...])
    h = _silu(_bdot(h, w2[...]) + b2[...])
    h = _silu(_bdot(h, w3[...]) + b3[...])
    h = _lrelu(_bdot(h, w4[...]) + b4[...])
    out[...] = _bdot(h, w5[...]) + b5[...]


def _head_mlp(g1, g2, ef, w1, b1, w2, b2, w3, b3, w4, b4, w5, b5):
    nrows = g1.shape[0]
    be = 2048
    no2 = NO * NO
    full = lambda a, b: pl.BlockSpec((a, b), lambda i: (0, 0))
    return pl.pallas_call(
        _head_body,
        grid=(nrows // be,),
        in_specs=[
            pl.BlockSpec((be, H), lambda i: (i, 0)),
            pl.BlockSpec((be, H), lambda i: (i, 0)),
            pl.BlockSpec((be, EC), lambda i: (i, 0)),
            full(EC, H), full(1, H),
            full(H, H), full(1, H),
            full(H, H), full(1, H),
            full(H, H), full(1, H),
            full(H, no2), full(1, no2),
        ],
        out_specs=pl.BlockSpec((be, no2), lambda i: (i, 0)),
        out_shape=jax.ShapeDtypeStruct((nrows, no2), _f32),
    )(g1, g2, ef, w1, b1, w2, b2, w3, b3, w4, b4, w5, b5)


def _make_gather2_body(rpt):
    ngsup = rpt // GSUP

    def body(ta, tb, ia, ib, oa, ob, iva, ivb, buf0, buf1,
             gsem0, gsem1, osem0, osem1):
        c = lax.axis_index("c")
        s = lax.axis_index("s")
        base = (c * NS + s) * rpt
        pltpu.sync_copy(ia.at[pl.ds(base, rpt)], iva)
        pltpu.sync_copy(ib.at[pl.ds(base, rpt)], ivb)

        bufs = (buf0, buf1)
        gsems = (gsem0, gsem1)
        osems = (osem0, osem1)
        chunks = [(ta, iva, oa, u) for u in range(ngsup)] + \
                 [(tb, ivb, ob, u) for u in range(ngsup)]
        nk = len(chunks)
        ghandles = {}
        ohandles = {}

        def flush(k):
            pb = k & 1
            for h in ghandles[k]:
                h.wait()
            _, _, pout, pu = chunks[k]
            ohandles[k] = pltpu.async_copy(
                bufs[pb], pout.at[pl.ds(base + pu * GSUP, GSUP)], osems[pb])

        for k, (tab, iv, out, u) in enumerate(chunks):
            b = k & 1
            if k >= 2:
                ohandles[k - 2].wait()
            offs = u * GSUP
            ghandles[k] = [
                pltpu.async_copy(
                    tab.at[iv.at[pl.ds(offs + j * SUB, SUB)]],
                    bufs[b].at[pl.ds(j * SUB, SUB)], gsems[b])
                for j in range(GSUB)
            ]
            if k >= 1:
                flush(k - 1)
        flush(nk - 1)
        ohandles[nk - 2].wait()
        ohandles[nk - 1].wait()

    return body


def _gather2(ta, tb, ia, ib):
    nrows = ia.shape[0]
    rpt = nrows // NW
    mesh = plsc.VectorSubcoreMesh(core_axis_name="c", subcore_axis_name="s")
    k = pl.kernel(
        _make_gather2_body(rpt),
        out_type=(jax.ShapeDtypeStruct((nrows, H), _bf16),
                  jax.ShapeDtypeStruct((nrows, H), _bf16)),
        mesh=mesh,
        compiler_params=pltpu.CompilerParams(use_tc_tiling_on_sc=False),
        scratch_types=[
            pltpu.VMEM((rpt,), jnp.int32),
            pltpu.VMEM((rpt,), jnp.int32),
            pltpu.VMEM((GSUP, H), _bf16),
            pltpu.VMEM((GSUP, H), _bf16),
            pltpu.SemaphoreType.DMA,
            pltpu.SemaphoreType.DMA,
            pltpu.SemaphoreType.DMA,
            pltpu.SemaphoreType.DMA,
        ],
    )
    return k(ta, tb, ia, ib)


def _make_scatter_body(rpt):
    nsub = SSUP // SUB
    nsup = rpt // SSUP
    nchk = rpt // SUB

    def body(eu, idx2, zero, out, idxv, rows, shared):
        c = lax.axis_index("c")
        s = lax.axis_index("s")
        wid = c * NS + s
        pltpu.sync_copy(zero.at[pl.ds(s * RPC, RPC)],
                        shared.at[pl.ds(s * RPC, RPC)])
        pltpu.sync_copy(idx2.at[pl.ds(wid * nchk, nchk)], idxv)
        plsc.subcore_barrier()

        def step(u, carry):
            pltpu.sync_copy(eu.at[pl.ds(wid * rpt + u * SSUP, SSUP)], rows)
            for j in range(nsub):
                pltpu.sync_copy(rows.at[pl.ds(j * SUB, SUB)],
                                shared.at[idxv.at[u * nsub + j]], add=True)
            return carry

        lax.fori_loop(0, nsup, step, 0)
        plsc.subcore_barrier()
        pltpu.sync_copy(shared.at[pl.ds(s * RPC, RPC)],
                        out.at[pl.ds(c * NP + s * RPC, RPC)])

    return body


def _scatter_add(eu, idx2, zero):
    rpt = eu.shape[0] // NW
    mesh = plsc.VectorSubcoreMesh(core_axis_name="c", subcore_axis_name="s")
    k = pl.kernel(
        _make_scatter_body(rpt),
        out_type=jax.ShapeDtypeStruct((NC * NP, EC), _f32),
        mesh=mesh,
        compiler_params=pltpu.CompilerParams(use_tc_tiling_on_sc=False),
        scratch_types=[
            pltpu.VMEM((rpt // SUB, SUB), jnp.int32),
            pltpu.VMEM((SSUP, EC), _f32),
            pltpu.VMEM_SHARED((NP, EC), _f32),
        ],
    )
    return k(eu, idx2, zero)


def kernel(node_env, edge_radial, edge_angular, edge_index,
           nu_w1, nu_b1, nu_w2, nu_b2,
           eu_w1, eu_b1, eu_w2, eu_b2, eu_w3, eu_b3, eu_w4, eu_b4,
           eu_w5, eu_b5, eu_w6, eu_b6,
           h_w1, h_b1, h_w2, h_b2, h_w3, h_b3, h_w4, h_b4, h_w5, h_b5):
    rd = edge_radial.shape[1]
    ed = rd + edge_angular.shape[1]
    pad_e = EPAD - E

    src = jnp.concatenate([edge_index[0],
                           jnp.zeros((pad_e,), jnp.int32)])
    dst = jnp.concatenate([edge_index[1],
                           jnp.full((pad_e,), N, jnp.int32)])

    ef = jnp.concatenate(
        [edge_radial, edge_angular,
         jnp.zeros((E, EC - ed), _f32)], axis=1)
    ef = jnp.concatenate([ef, jnp.zeros((pad_e, EC), _f32)], axis=0)

    def pad_rows(w):
        return jnp.concatenate(
            [w, jnp.zeros((EC - w.shape[0], w.shape[1]), _f32)], axis=0)

    eu1s, eu1d = eu_w1[:D], eu_w1[D:2 * D]
    eu1e = pad_rows(eu_w1[2 * D:])
    h1s, h1d = h_w1[:D], h_w1[D:2 * D]
    h1e = pad_rows(h_w1[2 * D:])
    nu1n = nu_w1[:D]
    nu1a = pad_rows(nu_w1[D:])
    eu_w6p = jnp.concatenate(
        [eu_w6, jnp.zeros((H, EC - eu_w6.shape[1]), _f32)], axis=1)
    eu_b6p = jnp.concatenate([eu_b6, jnp.zeros((EC - eu_b6.shape[0],), _f32)])

    r1 = lambda b: b.reshape(1, -1)
    zero_np = jnp.zeros((NP, EC), _f32)

    srcs = [lax.slice(src, (i * EH,), ((i + 1) * EH,)) for i in range(NSPLIT)]
    dsts = [lax.slice(dst, (i * EH,), ((i + 1) * EH,)) for i in range(NSPLIT)]
    dst2s = [d.reshape(EH // SUB, SUB) for d in dsts]
    hbs = [i * (EH // 2048) for i in range(NSPLIT)]

    p1s, p1d = _node_proj(node_env, eu1s, eu1d)
    gs = [_gather2(p1s, p1d, srcs[i], dsts[i]) for i in range(NSPLIT)]
    e_upds = [_edge_mlp(gs[i][0], gs[i][1], ef, hbs[i],
                        eu1e, r1(eu_b1), eu_w2, r1(eu_b2), eu_w3, r1(eu_b3),
                        eu_w4, r1(eu_b4), eu_w5, r1(eu_b5),
                        eu_w6p, r1(eu_b6p))
              for i in range(NSPLIT)]
    aggs = [_scatter_add(e_upds[i], dst2s[i], zero_np).reshape(NC, NP, EC)
            for i in range(NSPLIT)]
    p2s, p2d = _node_update(node_env, aggs,
                            nu1n, nu1a, r1(nu_b1), nu_w2, r1(nu_b2),
                            h1s, h1d)
    g2s = [_gather2(p2s, p2d, srcs[i], dsts[i]) for i in range(NSPLIT)]
    outs = [_head_mlp(g2s[i][0], g2s[i][1], e_upds[i],
                      h1e, r1(h_b1), h_w2, r1(h_b2), h_w3, r1(h_b3),
                      h_w4, r1(h_b4), h_w5, r1(h_b5))
            for i in range(NSPLIT)]
    out = jnp.concatenate(outs, axis=0)
    return out[:E].reshape(E, NO, NO)


# speedup vs baseline: 1.4113x; 1.0083x over previous
import functools

import jax
import jax.numpy as jnp
from jax import lax
from jax.experimental import pallas as pl
from jax.experimental.pallas import tpu as pltpu
from jax.experimental.pallas import tpu_sc as plsc

N = 10000
E = 160000
D = 128
H = 64
NO = 4

NC = 2
NS = 16
NW = NC * NS

EPAD = 163840
NSPLIT = 2
EH = EPAD // NSPLIT

SUB = 128
SSUP = 640
GSUP = 640
GSUB = GSUP // SUB
NP = 10240
EC = 32
RPC = NP // NS

_f32 = jnp.float32
_bf16 = jnp.bfloat16


def _silu(x):
    return x * jax.nn.sigmoid(x)


def _lrelu(x):
    return jnp.where(x >= 0, x, 0.01 * x)


def _bdot(x, w):
    return jnp.dot(x.astype(_bf16), w.astype(_bf16),
                   preferred_element_type=_f32)


def _node_proj_body(ne, ws, wd, o1, o2):
    x = ne[...]
    o1[...] = jnp.dot(x, ws[...], preferred_element_type=_f32).astype(_bf16)
    o2[...] = jnp.dot(x, wd[...], preferred_element_type=_f32).astype(_bf16)


def _node_proj(ne, ws, wd):
    bn = 1000
    return pl.pallas_call(
        _node_proj_body,
        grid=(N // bn,),
        in_specs=[
            pl.BlockSpec((bn, D), lambda i: (i, 0)),
            pl.BlockSpec((D, H), lambda i: (0, 0)),
            pl.BlockSpec((D, H), lambda i: (0, 0)),
        ],
        out_specs=[pl.BlockSpec((bn, H), lambda i: (i, 0))] * 2,
        out_shape=[jax.ShapeDtypeStruct((N, H), _bf16)] * 2,
    )(ne, ws, wd)


def _edge_mlp_body(g1, g2, ef, w1, b1, w2, b2, w3, b3, w4, b4, w5, b5,
                   w6, b6, out):
    h = _silu(g1[...].astype(_f32) + g2[...].astype(_f32)
              + _bdot(ef[...], w1[...]) + b1[...])
    h = _silu(_bdot(h, w2[...]) + b2[...])
    h = _silu(_bdot(h, w3[...]) + b3[...])
    h = _silu(_bdot(h, w4[...]) + b4[...])
    h = _lrelu(_bdot(h, w5[...]) + b5[...])
    out[...] = _bdot(h, w6[...]) + b6[...]


def _edge_mlp(g1, g2, ef, hb, w1, b1, w2, b2, w3, b3, w4, b4, w5, b5,
              w6, b6):
    nrows = g1.shape[0]
    be = 2048
    full = lambda a, b: pl.BlockSpec((a, b), lambda i: (0, 0))
    return pl.pallas_call(
        _edge_mlp_body,
        grid=(nrows // be,),
        in_specs=[
            pl.BlockSpec((be, H), lambda i: (i, 0)),
            pl.BlockSpec((be, H), lambda i: (i, 0)),
            pl.BlockSpec((be, EC), lambda i, hb=hb: (i + hb, 0)),
            full(EC, H), full(1, H),
            full(H, H), full(1, H),
            full(H, H // 2), full(1, H // 2),
            full(H // 2, H), full(1, H),
            full(H, H), full(1, H),
            full(H, EC), full(1, EC),
        ],
        out_specs=pl.BlockSpec((be, EC), lambda i: (i, 0)),
        out_shape=jax.ShapeDtypeStruct((nrows, EC), _f32),
    )(g1, g2, ef, w1, b1, w2, b2, w3, b3, w4, b4, w5, b5, w6, b6)


def _node_update_body(ne, agg_refs, wn, wa, b1, w2, b2, hs, hd, o1, o2):
    a = agg_refs[0][0] + agg_refs[0][1]
    for r in agg_refs[1:]:
        a = a + r[0] + r[1]
    h = _silu(jnp.dot(ne[...], wn[...], preferred_element_type=_f32)
              + jnp.dot(a, wa[...], preferred_element_type=_f32)
              + b1[...])
    nu = jnp.dot(h, w2[...], preferred_element_type=_f32) + b2[...]
    o1[...] = jnp.dot(nu, hs[...], preferred_element_type=_f32).astype(_bf16)
    o2[...] = jnp.dot(nu, hd[...], preferred_element_type=_f32).astype(_bf16)


def _node_update(ne, aggs, wn, wa, b1, w2, b2, hs, hd):
    bn = 1000
    full = lambda a, b: pl.BlockSpec((a, b), lambda i: (0, 0))

    def body(ne_, *rest):
        agg_refs = rest[:NSPLIT]
        (wn_, wa_, b1_, w2_, b2_, hs_, hd_, o1_, o2_) = rest[NSPLIT:]
        _node_update_body(ne_, agg_refs, wn_, wa_, b1_, w2_, b2_, hs_, hd_,
                          o1_, o2_)

    return pl.pallas_call(
        body,
        grid=(N // bn,),
        in_specs=[pl.BlockSpec((bn, D), lambda i: (i, 0))]
        + [pl.BlockSpec((NC, bn, EC), lambda i: (0, i, 0))] * NSPLIT
        + [full(D, H), full(EC, H), full(1, H),
           full(H, D), full(1, D),
           full(D, H), full(D, H)],
        out_specs=[pl.BlockSpec((bn, H), lambda i: (i, 0))] * 2,
        out_shape=[jax.ShapeDtypeStruct((N, H), _bf16)] * 2,
    )(ne, *aggs, wn, wa, b1, w2, b2, hs, hd)


def _head_body(g1, g2, ef, w1, b1, w2, b2, w3, b3, w4, b4, w5, b5, out):
    h = _silu(g1[...].astype(_f32) + g2[...].astype(_f32)
              + _bdot(ef[...], w1[...]) + b1[...])
    h = _silu(_bdot(h, w2[...]) + b2[...])
    h = _silu(_bdot(h, w3[...]) + b3[...])
    h = _lrelu(_bdot(h, w4[...]) + b4[...])
    out[...] = _bdot(h, w5[...]) + b5[...]


def _head_mlp(g1, g2, ef, w1, b1, w2, b2, w3, b3, w4, b4, w5, b5):
    nrows = g1.shape[0]
    be = 2048
    no2 = NO * NO
    full = lambda a, b: pl.BlockSpec((a, b), lambda i: (0, 0))
    return pl.pallas_call(
        _head_body,
        grid=(nrows // be,),
        in_specs=[
            pl.BlockSpec((be, H), lambda i: (i, 0)),
            pl.BlockSpec((be, H), lambda i: (i, 0)),
            pl.BlockSpec((be, EC), lambda i: (i, 0)),
            full(EC, H), full(1, H),
            full(H, H), full(1, H),
            full(H, H), full(1, H),
            full(H, H), full(1, H),
            full(H, no2), full(1, no2),
        ],
        out_specs=pl.BlockSpec((be, no2), lambda i: (i, 0)),
        out_shape=jax.ShapeDtypeStruct((nrows, no2), _f32),
    )(g1, g2, ef, w1, b1, w2, b2, w3, b3, w4, b4, w5, b5)


def _make_gather2_body(rpt, ho):
    ngsup = rpt // GSUP

    def body(ta, tb, ia, ib, oa, ob, iva, ivb, buf0, buf1,
             gsem0, gsem1, osem0, osem1):
        c = lax.axis_index("c")
        s = lax.axis_index("s")
        base = (c * NS + s) * rpt
        pltpu.sync_copy(ia.at[pl.ds(ho + base, rpt)], iva)
        pltpu.sync_copy(ib.at[pl.ds(ho + base, rpt)], ivb)

        bufs = (buf0, buf1)
        gsems = (gsem0, gsem1)
        osems = (osem0, osem1)
        chunks = [(ta, iva, oa, u) for u in range(ngsup)] + \
                 [(tb, ivb, ob, u) for u in range(ngsup)]
        nk = len(chunks)
        ghandles = {}
        ohandles = {}

        def flush(k):
            pb = k & 1
            for h in ghandles[k]:
                h.wait()
            _, _, pout, pu = chunks[k]
            ohandles[k] = pltpu.async_copy(
                bufs[pb], pout.at[pl.ds(base + pu * GSUP, GSUP)], osems[pb])

        for k, (tab, iv, out, u) in enumerate(chunks):
            b = k & 1
            if k >= 2:
                ohandles[k - 2].wait()
            offs = u * GSUP
            ghandles[k] = [
                pltpu.async_copy(
                    tab.at[iv.at[pl.ds(offs + j * SUB, SUB)]],
                    bufs[b].at[pl.ds(j * SUB, SUB)], gsems[b])
                for j in range(GSUB)
            ]
            if k >= 1:
                flush(k - 1)
        flush(nk - 1)
        ohandles[nk - 2].wait()
        ohandles[nk - 1].wait()

    return body


def _gather2(ta, tb, ia, ib, ho):
    nrows = EH
    rpt = nrows // NW
    mesh = plsc.VectorSubcoreMesh(core_axis_name="c", subcore_axis_name="s")
    k = pl.kernel(
        _make_gather2_body(rpt, ho),
        out_type=(jax.ShapeDtypeStruct((nrows, H), _bf16),
                  jax.ShapeDtypeStruct((nrows, H), _bf16)),
        mesh=mesh,
        compiler_params=pltpu.CompilerParams(use_tc_tiling_on_sc=False),
        scratch_types=[
            pltpu.VMEM((rpt,), jnp.int32),
            pltpu.VMEM((rpt,), jnp.int32),
            pltpu.VMEM((GSUP, H), _bf16),
            pltpu.VMEM((GSUP, H), _bf16),
            pltpu.SemaphoreType.DMA,
            pltpu.SemaphoreType.DMA,
            pltpu.SemaphoreType.DMA,
            pltpu.SemaphoreType.DMA,
        ],
    )
    return k(ta, tb, ia, ib)


def _make_scatter_body(rpt, ho):
    nsub = SSUP // SUB
    nsup = rpt // SSUP
    nchk = rpt // SUB

    def body(eu, idx2, zero, out, idxv, rows, shared):
        c = lax.axis_index("c")
        s = lax.axis_index("s")
        wid = c * NS + s
        pltpu.sync_copy(zero.at[pl.ds(s * RPC, RPC)],
                        shared.at[pl.ds(s * RPC, RPC)])
        pltpu.sync_copy(idx2.at[pl.ds(ho // SUB + wid * nchk, nchk)], idxv)
        plsc.subcore_barrier()

        def step(u, carry):
            pltpu.sync_copy(eu.at[pl.ds(wid * rpt + u * SSUP, SSUP)], rows)
            for j in range(nsub):
                pltpu.sync_copy(rows.at[pl.ds(j * SUB, SUB)],
                                shared.at[idxv.at[u * nsub + j]], add=True)
            return carry

        lax.fori_loop(0, nsup, step, 0)
        plsc.subcore_barrier()
        pltpu.sync_copy(shared.at[pl.ds(s * RPC, RPC)],
                        out.at[pl.ds(c * NP + s * RPC, RPC)])

    return body


def _scatter_add(eu, idx2, zero, ho):
    rpt = eu.shape[0] // NW
    mesh = plsc.VectorSubcoreMesh(core_axis_name="c", subcore_axis_name="s")
    k = pl.kernel(
        _make_scatter_body(rpt, ho),
        out_type=jax.ShapeDtypeStruct((NC * NP, EC), _f32),
        mesh=mesh,
        compiler_params=pltpu.CompilerParams(use_tc_tiling_on_sc=False),
        scratch_types=[
            pltpu.VMEM((rpt // SUB, SUB), jnp.int32),
            pltpu.VMEM((SSUP, EC), _f32),
            pltpu.VMEM_SHARED((NP, EC), _f32),
        ],
    )
    return k(eu, idx2, zero)


def kernel(node_env, edge_radial, edge_angular, edge_index,
           nu_w1, nu_b1, nu_w2, nu_b2,
           eu_w1, eu_b1, eu_w2, eu_b2, eu_w3, eu_b3, eu_w4, eu_b4,
           eu_w5, eu_b5, eu_w6, eu_b6,
           h_w1, h_b1, h_w2, h_b2, h_w3, h_b3, h_w4, h_b4, h_w5, h_b5):
    rd = edge_radial.shape[1]
    ed = rd + edge_angular.shape[1]
    pad_e = EPAD - E

    src = jnp.concatenate([edge_index[0],
                           jnp.zeros((pad_e,), jnp.int32)])
    dst = jnp.concatenate([edge_index[1],
                           jnp.full((pad_e,), N, jnp.int32)])

    ef = jnp.concatenate(
        [edge_radial, edge_angular,
         jnp.zeros((E, EC - ed), _f32)], axis=1)
    ef = jnp.concatenate([ef, jnp.zeros((pad_e, EC), _f32)], axis=0)

    def pad_rows(w):
        return jnp.concatenate(
            [w, jnp.zeros((EC - w.shape[0], w.shape[1]), _f32)], axis=0)

    eu1s, eu1d = eu_w1[:D], eu_w1[D:2 * D]
    eu1e = pad_rows(eu_w1[2 * D:])
    h1s, h1d = h_w1[:D], h_w1[D:2 * D]
    h1e = pad_rows(h_w1[2 * D:])
    nu1n = nu_w1[:D]
    nu1a = pad_rows(nu_w1[D:])
    eu_w6p = jnp.concatenate(
        [eu_w6, jnp.zeros((H, EC - eu_w6.shape[1]), _f32)], axis=1)
    eu_b6p = jnp.concatenate([eu_b6, jnp.zeros((EC - eu_b6.shape[0],), _f32)])

    r1 = lambda b: b.reshape(1, -1)
    zero_np = jnp.zeros((NP, EC), _f32)

    dst2 = dst.reshape(EPAD // SUB, SUB)
    hos = [i * EH for i in range(NSPLIT)]
    hbs = [i * (EH // 2048) for i in range(NSPLIT)]

    p1s, p1d = _node_proj(node_env, eu1s, eu1d)
    gs = [_gather2(p1s, p1d, src, dst, hos[i]) for i in range(NSPLIT)]
    e_upds = [_edge_mlp(gs[i][0], gs[i][1], ef, hbs[i],
                        eu1e, r1(eu_b1), eu_w2, r1(eu_b2), eu_w3, r1(eu_b3),
                        eu_w4, r1(eu_b4), eu_w5, r1(eu_b5),
                        eu_w6p, r1(eu_b6p))
              for i in range(NSPLIT)]
    aggs = [_scatter_add(e_upds[i], dst2, zero_np, hos[i]).reshape(NC, NP, EC)
            for i in range(NSPLIT)]
    p2s, p2d = _node_update(node_env, aggs,
                            nu1n, nu1a, r1(nu_b1), nu_w2, r1(nu_b2),
                            h1s, h1d)
    g2s = [_gather2(p2s, p2d, src, dst, hos[i]) for i in range(NSPLIT)]
    outs = [_head_mlp(g2s[i][0], g2s[i][1], e_upds[i],
                      h1e, r1(h_b1), h_w2, r1(h_b2), h_w3, r1(h_b3),
                      h_w4, r1(h_b4), h_w5, r1(h_b5))
            for i in range(NSPLIT)]
    out = jnp.concatenate(outs, axis=0)
    return out[:E].reshape(E, NO, NO)
